# Initial kernel scaffold; baseline (speedup 1.0000x reference)
import jax
import jax.numpy as jnp
from jax.experimental import pallas as pl


def kernel(sequence_output, span_starts, k, entity_anchor, relation_embeddings, nota_embeddings):
    raise NotImplementedError("write your pallas kernel here")


# speedup vs baseline: 4.3859x; 4.3859x over previous
import jax
import jax.numpy as jnp
from jax import lax
from jax.experimental import pallas as pl
from jax.experimental.pallas import tpu as pltpu
from jax.experimental.pallas import tpu_sc as plsc

SPAN_LEN = 4
B = 4
T = 2048
H = 768
N_SPANS = 1024
TOPK = 50
DPAD = 128

_NC = 2
_NS = 16
_NW = _NC * _NS
_SPANS_PER_W = (B * N_SPANS) // _NW
_CH = 32


def _sc_gather_body(seq_hbm, starts_hbm, out_hbm,
                    sidx, idx0, idx1, idx2, idx3,
                    buf0, buf1, buf2, buf3, gsem):
    wid = lax.axis_index("s") * _NC + lax.axis_index("c")
    base = wid * _SPANS_PER_W
    boff = (base // N_SPANS) * T
    idxs = (idx0, idx1, idx2, idx3)
    bufs = (buf0, buf1, buf2, buf3)
    for c in range(_SPANS_PER_W // _CH):
        sp = base + c * _CH
        pltpu.sync_copy(starts_hbm.at[pl.ds(sp, _CH)], sidx)
        for j in range(SPAN_LEN):
            for h in range(_CH // 16):
                sl = pl.ds(h * 16, 16)
                idxs[j][sl] = sidx[sl] + (boff + j)
        cps = [pltpu.async_copy(seq_hbm.at[idxs[j]], bufs[j], gsem)
               for j in range(SPAN_LEN)]
        for cp in cps:
            cp.wait()
        for j in range(SPAN_LEN):
            pltpu.sync_copy(bufs[j], out_hbm.at[j, pl.ds(sp, _CH)])


def _sc_gather(seq_flat, starts_flat):
    mesh = plsc.VectorSubcoreMesh(
        core_axis_name="c", subcore_axis_name="s",
        num_cores=_NC, num_subcores=_NS)
    return pl.kernel(
        _sc_gather_body,
        out_type=jax.ShapeDtypeStruct((SPAN_LEN, B * N_SPANS, H),
                                      jnp.float32),
        mesh=mesh,
        scratch_types=(
            [pltpu.VMEM((_CH,), jnp.int32) for _ in range(5)]
            + [pltpu.VMEM((_CH, H), jnp.float32) for _ in range(SPAN_LEN)]
            + [pltpu.SemaphoreType.DMA]
        ),
    )(seq_flat, starts_flat)


def _select_body(kd_ref, tok_ref, at_ref, wt_ref, out_ref, p48_ref, sel_ref):
    t0, t1, t2, t3 = tok_ref[0], tok_ref[1], tok_ref[2], tok_ref[3]
    embs = ((t0 + t2) + (t1 + t3)) * 0.25
    sc128 = jnp.dot(embs, at_ref[...],
                    preferred_element_type=jnp.float32)
    p48_ref[...] = jnp.dot(embs, wt_ref[...],
                           precision=jax.lax.Precision.HIGHEST,
                           preferred_element_type=jnp.float32)
    q3 = sc128.reshape(8, N_SPANS // 8, DPAD)
    sc = jnp.maximum(jnp.maximum(q3[:, :, 0], q3[:, :, 1]), q3[:, :, 2])
    flat = (lax.broadcasted_iota(jnp.int32, (8, N_SPANS // 8), 0)
            * (N_SPANS // 8)
            + lax.broadcasted_iota(jnp.int32, (8, N_SPANS // 8), 1))
    kd = kd_ref[0]
    neg = jnp.float32(-jnp.inf)
    for i in range(TOPK):
        m = jnp.max(sc)
        idx = jnp.min(jnp.where(sc == m, flat, jnp.int32(N_SPANS)))
        sel_ref[i, :] = p48_ref[pl.ds(idx + kd, 1), :][0]
        sc = jnp.where(flat == idx, neg, sc)
    s = sel_ref[...]
    a_rel = s[:, 0:4]
    b_rel = s[:, 4:8]
    a_nota = s[:, 8:28]
    b_nota = s[:, 28:48]
    rel = a_rel[:, None, :] + b_rel[None, :, :]
    nota = jnp.max(a_nota[:, None, :] + b_nota[None, :, :],
                   axis=-1, keepdims=True)
    out_ref[0, :, :, 0:1] = nota
    out_ref[0, :, :, 1:5] = rel
    out_ref[0, :, :, 5:8] = jnp.zeros((TOPK, TOPK, 3), jnp.float32)


def _select(kd, tok4, a_t, w_t):
    return pl.pallas_call(
        _select_body,
        grid=(B,),
        in_specs=[
            pl.BlockSpec(memory_space=pltpu.SMEM),
            pl.BlockSpec((SPAN_LEN, N_SPANS, H), lambda b: (0, b, 0)),
            pl.BlockSpec((H, DPAD), lambda b: (0, 0)),
            pl.BlockSpec((H, DPAD), lambda b: (0, 0)),
        ],
        out_specs=pl.BlockSpec((1, TOPK, TOPK, 8), lambda b: (b, 0, 0, 0)),
        out_shape=jax.ShapeDtypeStruct((B, TOPK, TOPK, 8), jnp.float32),
        scratch_shapes=[
            pltpu.VMEM((N_SPANS, DPAD), jnp.float32),
            pltpu.VMEM((TOPK, DPAD), jnp.float32),
        ],
        compiler_params=pltpu.CompilerParams(
            dimension_semantics=("arbitrary",)),
    )(kd, tok4, a_t, w_t)


def kernel(sequence_output, span_starts, k, entity_anchor,
           relation_embeddings, nota_embeddings):
    a_t = jnp.zeros((H, DPAD), jnp.float32).at[:, :3].set(entity_anchor.T)
    w_t = jnp.concatenate([
        relation_embeddings[:, :H].T,
        relation_embeddings[:, H:].T,
        nota_embeddings[:, :H].T,
        nota_embeddings[:, H:].T,
        jnp.zeros((H, DPAD - 48), jnp.float32),
    ], axis=1)

    tok4 = _sc_gather(sequence_output.reshape(B * T, H),
                      span_starts.reshape(-1))

    kd = (jnp.asarray(k, jnp.int32) - TOPK).reshape(1)
    out8 = _select(kd, tok4, a_t, w_t)
    return out8[..., :5]


# speedup vs baseline: 6.1479x; 1.4017x over previous
import jax
import jax.numpy as jnp
from jax import lax
from jax.experimental import pallas as pl
from jax.experimental.pallas import tpu as pltpu
from jax.experimental.pallas import tpu_sc as plsc

SPAN_LEN = 4
B = 4
T = 2048
H = 768
N_SPANS = 1024
TOPK = 50
DPAD = 128

_NC = 2
_NS = 16
_NW = _NC * _NS
_SPANS_PER_W = (B * N_SPANS) // _NW
_CH = 16
_NCH = _SPANS_PER_W // _CH


def _sc_gather_body(seq_hbm, starts_hbm, out_hbm,
                    sidx, idx_v, buf_v, gsem0, gsem1, ssem0, ssem1):
    wid = lax.axis_index("s") * _NC + lax.axis_index("c")
    base = wid * _SPANS_PER_W
    boff = (base // N_SPANS) * T
    gsems = (gsem0, gsem1)
    ssems = (ssem0, ssem1)

    def stage(c, p):
        sp = base + c * _CH
        pltpu.sync_copy(starts_hbm.at[pl.ds(sp, _CH)], sidx)
        for j in range(SPAN_LEN):
            idx_v[p, j] = sidx[...] + (boff + j)
        return [pltpu.async_copy(seq_hbm.at[idx_v.at[p, j]],
                                 buf_v.at[p, j], gsems[p])
                for j in range(SPAN_LEN)]

    gcps = {0: stage(0, 0)}
    scps = {}
    for c in range(_NCH):
        p = c % 2
        q = 1 - p
        for cp in gcps.pop(c):
            cp.wait()
        sp = base + c * _CH
        scps[c] = [pltpu.async_copy(buf_v.at[p, j],
                                    out_hbm.at[j, pl.ds(sp, _CH)], ssems[p])
                   for j in range(SPAN_LEN)]
        if c + 1 < _NCH:
            if c >= 1:
                for cp in scps.pop(c - 1):
                    cp.wait()
            gcps[c + 1] = stage(c + 1, q)
    for c in sorted(scps):
        for cp in scps[c]:
            cp.wait()


def _sc_gather(seq_flat, starts_flat):
    mesh = plsc.VectorSubcoreMesh(
        core_axis_name="c", subcore_axis_name="s",
        num_cores=_NC, num_subcores=_NS)
    return pl.kernel(
        _sc_gather_body,
        out_type=jax.ShapeDtypeStruct((SPAN_LEN, B * N_SPANS, H),
                                      jnp.float32),
        mesh=mesh,
        scratch_types=(
            [pltpu.VMEM((_CH,), jnp.int32),
             pltpu.VMEM((2, SPAN_LEN, _CH), jnp.int32),
             pltpu.VMEM((2, SPAN_LEN, _CH, H), jnp.float32)]
            + [pltpu.SemaphoreType.DMA] * 4
        ),
    )(seq_flat, starts_flat)


def _score_body(tok_ref, at_ref, wt_ref, sc_ref, p48_ref):
    t0, t1, t2, t3 = tok_ref[0], tok_ref[1], tok_ref[2], tok_ref[3]
    embs = ((t0 + t2) + (t1 + t3)) * 0.25
    sc128 = jnp.dot(embs, at_ref[...],
                    preferred_element_type=jnp.float32)
    p48_ref[0] = jnp.dot(embs, wt_ref[...],
                         precision=jax.lax.Precision.HIGHEST,
                         preferred_element_type=jnp.float32)
    q3 = sc128.reshape(8, N_SPANS // 8, DPAD)
    sc_ref[0] = jnp.maximum(jnp.maximum(q3[:, :, 0], q3[:, :, 1]),
                            q3[:, :, 2])


def _score(tok4, a_t, w_t):
    return pl.pallas_call(
        _score_body,
        grid=(B,),
        in_specs=[
            pl.BlockSpec((SPAN_LEN, N_SPANS, H), lambda b: (0, b, 0)),
            pl.BlockSpec((H, DPAD), lambda b: (0, 0)),
            pl.BlockSpec((H, DPAD), lambda b: (0, 0)),
        ],
        out_specs=[
            pl.BlockSpec((1, 8, N_SPANS // 8), lambda b: (b, 0, 0)),
            pl.BlockSpec((1, N_SPANS, DPAD), lambda b: (b, 0, 0)),
        ],
        out_shape=[
            jax.ShapeDtypeStruct((B, 8, N_SPANS // 8), jnp.float32),
            jax.ShapeDtypeStruct((B, N_SPANS, DPAD), jnp.float32),
        ],
        compiler_params=pltpu.CompilerParams(
            dimension_semantics=("arbitrary",)),
    )(tok4, a_t, w_t)


_SUBL = 8
_LANE = N_SPANS // 8


def _bitonic_sort_desc(s, si, flat_i):
    for lk in range(1, 11):
        k = 1 << lk
        desc = (flat_i & k) == 0
        for lj in range(lk - 1, -1, -1):
            j = 1 << lj
            bit = (flat_i & j) != 0
            if j < _LANE:
                ps = jnp.where(bit, pltpu.roll(s, j, 1),
                               pltpu.roll(s, _LANE - j, 1))
                pi = jnp.where(bit, pltpu.roll(si, j, 1),
                               pltpu.roll(si, _LANE - j, 1))
            else:
                d = j // _LANE
                ps = jnp.where(bit, pltpu.roll(s, d, 0),
                               pltpu.roll(s, _SUBL - d, 0))
                pi = jnp.where(bit, pltpu.roll(si, d, 0),
                               pltpu.roll(si, _SUBL - d, 0))
            gt = (ps > s) | ((ps == s) & (pi < si))
            take = jnp.logical_not(jnp.logical_xor(gt,
                                                   jnp.logical_xor(desc, bit)))
            s = jnp.where(take, ps, s)
            si = jnp.where(take, pi, si)
    return s, si


def _select_body(kd_ref, sc_ref, p48_ref, out_ref):
    flat_i = (lax.broadcasted_iota(jnp.int32, (_SUBL, _LANE), 0) * _LANE
              + lax.broadcasted_iota(jnp.int32, (_SUBL, _LANE), 1))
    flatf = flat_i.astype(jnp.float32)
    kdf = kd_ref[0].astype(jnp.float32)
    ncol = lax.broadcasted_iota(jnp.int32, (N_SPANS, 64), 0
                                ).astype(jnp.float32)
    for b in range(B):
        _, si = _bitonic_sort_desc(sc_ref[b], flatf, flat_i)
        top64 = si[0:1, 0:64] + kdf
        oht = (ncol == top64).astype(jnp.float32)
        sel = lax.dot_general(oht, p48_ref[b],
                              (((0,), (0,)), ((), ())),
                              precision=jax.lax.Precision.HIGHEST,
                              preferred_element_type=jnp.float32)
        s = sel[:TOPK]
        a_rel = s[:, 0:4]
        b_rel = s[:, 4:8]
        a_nota = s[:, 8:28]
        b_nota = s[:, 28:48]
        rel = a_rel[:, None, :] + b_rel[None, :, :]
        nota = jnp.max(a_nota[:, None, :] + b_nota[None, :, :],
                       axis=-1, keepdims=True)
        out_ref[b, :, :, 0:1] = nota
        out_ref[b, :, :, 1:5] = rel


def _select(kd, sc4, p48):
    return pl.pallas_call(
        _select_body,
        in_specs=[
            pl.BlockSpec(memory_space=pltpu.SMEM),
            pl.BlockSpec((B, 8, N_SPANS // 8), lambda: (0, 0, 0)),
            pl.BlockSpec((B, N_SPANS, DPAD), lambda: (0, 0, 0)),
        ],
        out_specs=pl.BlockSpec((B, TOPK, TOPK, 5), lambda: (0, 0, 0, 0)),
        out_shape=jax.ShapeDtypeStruct((B, TOPK, TOPK, 5), jnp.float32),
    )(kd, sc4, p48)


def kernel(sequence_output, span_starts, k, entity_anchor,
           relation_embeddings, nota_embeddings):
    a_t = jnp.zeros((H, DPAD), jnp.float32).at[:, :3].set(entity_anchor.T)
    w_t = jnp.concatenate([
        relation_embeddings[:, :H].T,
        relation_embeddings[:, H:].T,
        nota_embeddings[:, :H].T,
        nota_embeddings[:, H:].T,
        jnp.zeros((H, DPAD - 48), jnp.float32),
    ], axis=1)

    tok4 = _sc_gather(sequence_output.reshape(B * T, H),
                      span_starts.reshape(-1))

    sc4, p48 = _score(tok4, a_t, w_t)
    kd = (jnp.asarray(k, jnp.int32) - TOPK).reshape(1)
    return _select(kd, sc4, p48)


# speedup vs baseline: 6.3486x; 1.0326x over previous
import jax
import jax.numpy as jnp
from jax import lax
from jax.experimental import pallas as pl
from jax.experimental.pallas import tpu as pltpu
from jax.experimental.pallas import tpu_sc as plsc

SPAN_LEN = 4
B = 4
T = 2048
H = 768
N_SPANS = 1024
TOPK = 50
DPAD = 128

_NC = 2
_NS = 16
_NW = _NC * _NS
_SPANS_PER_W = (B * N_SPANS) // _NW
_CH = 16
_NCH = _SPANS_PER_W // _CH


def _sc_gather_body(seq_hbm, starts_hbm, out_hbm,
                    sidx, idx_v, buf_v, gsem0, gsem1, ssem0, ssem1):
    wid = lax.axis_index("s") * _NC + lax.axis_index("c")
    base = wid * _SPANS_PER_W
    boff = (base // N_SPANS) * T
    gsems = (gsem0, gsem1)
    ssems = (ssem0, ssem1)

    def stage(c, p):
        sp = base + c * _CH
        pltpu.sync_copy(starts_hbm.at[pl.ds(sp, _CH)], sidx)
        for j in range(SPAN_LEN):
            idx_v[p, j] = sidx[...] + (boff + j)
        return [pltpu.async_copy(seq_hbm.at[idx_v.at[p, j]],
                                 buf_v.at[p, j], gsems[p])
                for j in range(SPAN_LEN)]

    gcps = {0: stage(0, 0)}
    scps = {}
    for c in range(_NCH):
        p = c % 2
        q = 1 - p
        for cp in gcps.pop(c):
            cp.wait()
        sp = base + c * _CH
        scps[c] = [pltpu.async_copy(buf_v.at[p, j],
                                    out_hbm.at[j, pl.ds(sp, _CH)], ssems[p])
                   for j in range(SPAN_LEN)]
        if c + 1 < _NCH:
            if c >= 1:
                for cp in scps.pop(c - 1):
                    cp.wait()
            gcps[c + 1] = stage(c + 1, q)
    for c in sorted(scps):
        for cp in scps[c]:
            cp.wait()


def _sc_gather(seq_flat, starts_flat):
    mesh = plsc.VectorSubcoreMesh(
        core_axis_name="c", subcore_axis_name="s",
        num_cores=_NC, num_subcores=_NS)
    return pl.kernel(
        _sc_gather_body,
        out_type=jax.ShapeDtypeStruct((SPAN_LEN, B * N_SPANS, H),
                                      jnp.float32),
        mesh=mesh,
        scratch_types=(
            [pltpu.VMEM((_CH,), jnp.int32),
             pltpu.VMEM((2, SPAN_LEN, _CH), jnp.int32),
             pltpu.VMEM((2, SPAN_LEN, _CH, H), jnp.float32)]
            + [pltpu.SemaphoreType.DMA] * 4
        ),
    )(seq_flat, starts_flat)


_SUBL = 8
_LANE = N_SPANS // 8


def _bitonic_sort_desc(s, si, flat_i):
    for lk in range(1, 11):
        k = 1 << lk
        desc = (flat_i & k) == 0
        for lj in range(lk - 1, -1, -1):
            j = 1 << lj
            bit = (flat_i & j) != 0
            if j < _LANE:
                ps = jnp.where(bit, pltpu.roll(s, j, 1),
                               pltpu.roll(s, _LANE - j, 1))
                pi = jnp.where(bit, pltpu.roll(si, j, 1),
                               pltpu.roll(si, _LANE - j, 1))
            else:
                d = j // _LANE
                ps = jnp.where(bit, pltpu.roll(s, d, 0),
                               pltpu.roll(s, _SUBL - d, 0))
                pi = jnp.where(bit, pltpu.roll(si, d, 0),
                               pltpu.roll(si, _SUBL - d, 0))
            gt = (ps > s) | ((ps == s) & (pi < si))
            take = jnp.logical_not(jnp.logical_xor(gt,
                                                   jnp.logical_xor(desc, bit)))
            s = jnp.where(take, ps, s)
            si = jnp.where(take, pi, si)
    return s, si


def _fused_body(kd_ref, tok_ref, at_ref, wt_ref, out_ref, sc_s, p48_s):
    s = pl.program_id(0)

    @pl.when(s < B)
    def _score():
        t0, t1, t2, t3 = tok_ref[0], tok_ref[1], tok_ref[2], tok_ref[3]
        embs = ((t0 + t2) + (t1 + t3)) * 0.25
        sc128 = jnp.dot(embs, at_ref[...],
                        preferred_element_type=jnp.float32)
        p48_s[s] = jnp.dot(embs, wt_ref[...],
                           precision=jax.lax.Precision.HIGHEST,
                           preferred_element_type=jnp.float32)
        q3 = sc128.reshape(_SUBL, _LANE, DPAD)
        sc_s[s] = jnp.maximum(jnp.maximum(q3[:, :, 0], q3[:, :, 1]),
                              q3[:, :, 2])

    @pl.when(s == B)
    def _select():
        flat_i = (lax.broadcasted_iota(jnp.int32, (_SUBL, _LANE), 0) * _LANE
                  + lax.broadcasted_iota(jnp.int32, (_SUBL, _LANE), 1))
        flatf = flat_i.astype(jnp.float32)
        kdf = kd_ref[0].astype(jnp.float32)
        ncol = lax.broadcasted_iota(jnp.int32, (N_SPANS, 64), 0
                                    ).astype(jnp.float32)
        for b in range(B):
            _, si = _bitonic_sort_desc(sc_s[b], flatf, flat_i)
            top64 = si[0:1, 0:64] + kdf
            oht = (ncol == top64).astype(jnp.float32)
            sel = lax.dot_general(oht, p48_s[b],
                                  (((0,), (0,)), ((), ())),
                                  precision=jax.lax.Precision.HIGHEST,
                                  preferred_element_type=jnp.float32)
            sl = sel[:TOPK]
            a_rel = sl[:, 0:4]
            b_rel = sl[:, 4:8]
            a_nota = sl[:, 8:28]
            b_nota = sl[:, 28:48]
            rel = a_rel[:, None, :] + b_rel[None, :, :]
            nota = jnp.max(a_nota[:, None, :] + b_nota[None, :, :],
                           axis=-1, keepdims=True)
            out_ref[b, :, :, 0:1] = nota
            out_ref[b, :, :, 1:5] = rel


def _fused(kd, tok4, a_t, w_t):
    return pl.pallas_call(
        _fused_body,
        grid=(B + 1,),
        in_specs=[
            pl.BlockSpec(memory_space=pltpu.SMEM),
            pl.BlockSpec((SPAN_LEN, N_SPANS, H),
                         lambda s: (0, jnp.minimum(s, B - 1), 0)),
            pl.BlockSpec((H, DPAD), lambda s: (0, 0)),
            pl.BlockSpec((H, DPAD), lambda s: (0, 0)),
        ],
        out_specs=pl.BlockSpec((B, TOPK, TOPK, 5), lambda s: (0, 0, 0, 0)),
        out_shape=jax.ShapeDtypeStruct((B, TOPK, TOPK, 5), jnp.float32),
        scratch_shapes=[
            pltpu.VMEM((B, _SUBL, _LANE), jnp.float32),
            pltpu.VMEM((B, N_SPANS, DPAD), jnp.float32),
        ],
        compiler_params=pltpu.CompilerParams(
            dimension_semantics=("arbitrary",)),
    )(kd, tok4, a_t, w_t)


def kernel(sequence_output, span_starts, k, entity_anchor,
           relation_embeddings, nota_embeddings):
    a_t = jnp.zeros((H, DPAD), jnp.float32).at[:, :3].set(entity_anchor.T)
    w_t = jnp.concatenate([
        relation_embeddings[:, :H].T,
        relation_embeddings[:, H:].T,
        nota_embeddings[:, :H].T,
        nota_embeddings[:, H:].T,
        jnp.zeros((H, DPAD - 48), jnp.float32),
    ], axis=1)

    tok4 = _sc_gather(sequence_output.reshape(B * T, H),
                      span_starts.reshape(-1))

    kd = (jnp.asarray(k, jnp.int32) - TOPK).reshape(1)
    return _fused(kd, tok4, a_t, w_t)


# speedup vs baseline: 7.2209x; 1.1374x over previous
import jax
import jax.numpy as jnp
from jax import lax
from jax.experimental import pallas as pl
from jax.experimental.pallas import tpu as pltpu
from jax.experimental.pallas import tpu_sc as plsc

SPAN_LEN = 4
B = 4
T = 2048
H = 768
N_SPANS = 1024
TOPK = 50
DPAD = 128

_NC = 2
_NS = 16
_NW = _NC * _NS
_SPANS_PER_W = (B * N_SPANS) // _NW
_CH = 16
_NCH = _SPANS_PER_W // _CH


def _sc_gather_body(seq_hbm, starts_hbm, out_hbm,
                    sidx, idx_v, buf_v, emb_v, gsem0, gsem1, ssem0, ssem1):
    wid = lax.axis_index("s") * _NC + lax.axis_index("c")
    base = wid * _SPANS_PER_W
    boff = (base // N_SPANS) * T
    gsems = (gsem0, gsem1)
    ssems = (ssem0, ssem1)

    def stage(c, p):
        sp = base + c * _CH
        pltpu.sync_copy(starts_hbm.at[pl.ds(sp, _CH)], sidx)
        for j in range(SPAN_LEN):
            idx_v[p, j] = sidx[...] + (boff + j)
        return [pltpu.async_copy(seq_hbm.at[idx_v.at[p, j]],
                                 buf_v.at[p, j], gsems[p])
                for j in range(SPAN_LEN)]

    def pool(p):
        def col(g, _):
            sl = pl.ds(g * 16, 16)
            for r in range(_CH):
                t02 = buf_v[p, 0, r, sl] + buf_v[p, 2, r, sl]
                t13 = buf_v[p, 1, r, sl] + buf_v[p, 3, r, sl]
                emb_v[p, r, sl] = (t02 + t13) * 0.25
            return _
        lax.fori_loop(0, H // 16, col, 0)

    gcps = {0: stage(0, 0)}
    scps = {}
    for c in range(_NCH):
        p = c % 2
        q = 1 - p
        for cp in gcps.pop(c):
            cp.wait()
        if c + 1 < _NCH:
            gcps[c + 1] = stage(c + 1, q)
        if c >= 2:
            for cp in scps.pop(c - 2):
                cp.wait()
        pool(p)
        sp = base + c * _CH
        scps[c] = [pltpu.async_copy(emb_v.at[p],
                                    out_hbm.at[pl.ds(sp, _CH)], ssems[p])]
    for c in sorted(scps):
        for cp in scps[c]:
            cp.wait()


def _sc_gather(seq_flat, starts_flat):
    mesh = plsc.VectorSubcoreMesh(
        core_axis_name="c", subcore_axis_name="s",
        num_cores=_NC, num_subcores=_NS)
    return pl.kernel(
        _sc_gather_body,
        out_type=jax.ShapeDtypeStruct((B * N_SPANS, H), jnp.float32),
        mesh=mesh,
        scratch_types=(
            [pltpu.VMEM((_CH,), jnp.int32),
             pltpu.VMEM((2, SPAN_LEN, _CH), jnp.int32),
             pltpu.VMEM((2, SPAN_LEN, _CH, H), jnp.float32),
             pltpu.VMEM((2, _CH, H), jnp.float32)]
            + [pltpu.SemaphoreType.DMA] * 4
        ),
    )(seq_flat, starts_flat)


_SUBL = 8
_LANE = N_SPANS // 8


def _bitonic_sort_desc(s, si, flat_i):
    for lk in range(1, 11):
        k = 1 << lk
        desc = (flat_i & k) == 0
        for lj in range(lk - 1, -1, -1):
            j = 1 << lj
            bit = (flat_i & j) != 0
            if j < _LANE:
                ps = jnp.where(bit, pltpu.roll(s, j, 1),
                               pltpu.roll(s, _LANE - j, 1))
                pi = jnp.where(bit, pltpu.roll(si, j, 1),
                               pltpu.roll(si, _LANE - j, 1))
            else:
                d = j // _LANE
                ps = jnp.where(bit, pltpu.roll(s, d, 0),
                               pltpu.roll(s, _SUBL - d, 0))
                pi = jnp.where(bit, pltpu.roll(si, d, 0),
                               pltpu.roll(si, _SUBL - d, 0))
            gt = (ps > s) | ((ps == s) & (pi < si))
            take = jnp.logical_not(jnp.logical_xor(gt,
                                                   jnp.logical_xor(desc, bit)))
            s = jnp.where(take, ps, s)
            si = jnp.where(take, pi, si)
    return s, si


def _fused_body(kd_ref, tok_ref, at_ref, wt_ref, out_ref, sc_s, p48_s):
    s = pl.program_id(0)

    @pl.when(s < B)
    def _score():
        embs = tok_ref[0]
        sc128 = jnp.dot(embs, at_ref[...],
                        preferred_element_type=jnp.float32)
        p48_s[s] = jnp.dot(embs, wt_ref[...],
                           precision=jax.lax.Precision.HIGHEST,
                           preferred_element_type=jnp.float32)
        q3 = sc128.reshape(_SUBL, _LANE, DPAD)
        sc_s[s] = jnp.maximum(jnp.maximum(q3[:, :, 0], q3[:, :, 1]),
                              q3[:, :, 2])

    @pl.when(s == B)
    def _select():
        flat_i = (lax.broadcasted_iota(jnp.int32, (_SUBL, _LANE), 0) * _LANE
                  + lax.broadcasted_iota(jnp.int32, (_SUBL, _LANE), 1))
        flatf = flat_i.astype(jnp.float32)
        kdf = kd_ref[0].astype(jnp.float32)
        ncol = lax.broadcasted_iota(jnp.int32, (N_SPANS, 64), 0
                                    ).astype(jnp.float32)
        for b in range(B):
            _, si = _bitonic_sort_desc(sc_s[b], flatf, flat_i)
            top64 = si[0:1, 0:64] + kdf
            oht = (ncol == top64).astype(jnp.float32)
            sel = lax.dot_general(oht, p48_s[b],
                                  (((0,), (0,)), ((), ())),
                                  precision=jax.lax.Precision.HIGHEST,
                                  preferred_element_type=jnp.float32)
            sl = sel[:TOPK]
            a_rel = sl[:, 0:4]
            b_rel = sl[:, 4:8]
            a_nota = sl[:, 8:28]
            b_nota = sl[:, 28:48]
            rel = a_rel[:, None, :] + b_rel[None, :, :]
            nota = jnp.max(a_nota[:, None, :] + b_nota[None, :, :],
                           axis=-1, keepdims=True)
            out_ref[b, :, :, 0:1] = nota
            out_ref[b, :, :, 1:5] = rel


def _fused(kd, tok4, a_t, w_t):
    return pl.pallas_call(
        _fused_body,
        grid=(B + 1,),
        in_specs=[
            pl.BlockSpec(memory_space=pltpu.SMEM),
            pl.BlockSpec((1, N_SPANS, H),
                         lambda s: (jnp.minimum(s, B - 1), 0, 0)),
            pl.BlockSpec((H, DPAD), lambda s: (0, 0)),
            pl.BlockSpec((H, DPAD), lambda s: (0, 0)),
        ],
        out_specs=pl.BlockSpec((B, TOPK, TOPK, 5), lambda s: (0, 0, 0, 0)),
        out_shape=jax.ShapeDtypeStruct((B, TOPK, TOPK, 5), jnp.float32),
        scratch_shapes=[
            pltpu.VMEM((B, _SUBL, _LANE), jnp.float32),
            pltpu.VMEM((B, N_SPANS, DPAD), jnp.float32),
        ],
        compiler_params=pltpu.CompilerParams(
            dimension_semantics=("arbitrary",)),
    )(kd, tok4, a_t, w_t)


def kernel(sequence_output, span_starts, k, entity_anchor,
           relation_embeddings, nota_embeddings):
    a_t = jnp.zeros((H, DPAD), jnp.float32).at[:, :3].set(entity_anchor.T)
    w_t = jnp.concatenate([
        relation_embeddings[:, :H].T,
        relation_embeddings[:, H:].T,
        nota_embeddings[:, :H].T,
        nota_embeddings[:, H:].T,
        jnp.zeros((H, DPAD - 48), jnp.float32),
    ], axis=1)

    embs = _sc_gather(sequence_output.reshape(B * T, H),
                      span_starts.reshape(-1))

    kd = (jnp.asarray(k, jnp.int32) - TOPK).reshape(1)
    return _fused(kd, embs.reshape(B, N_SPANS, H), a_t, w_t)
